# baseline (speedup 1.0000x reference)
import functools

import jax
import jax.numpy as jnp
from jax.experimental import pallas as pl
from jax.experimental.pallas import tpu as pltpu


def _linear_kstream_kernel(x_ref, w_ref, b_ref, o_ref):
    k = pl.program_id(1)

    @pl.when(k == 0)
    def _():
        o_ref[...] = jnp.broadcast_to(b_ref[...], o_ref.shape)

    o_ref[...] += jnp.dot(
        x_ref[...], w_ref[...], preferred_element_type=jnp.float32
    )


@functools.partial(jax.jit, static_argnames=("num_m", "num_k"))
def _linear(x2d, weight, bias, *, num_m, num_k):
    m, k = x2d.shape
    _, n = weight.shape
    bm, bk = m // num_m, k // num_k

    return pl.pallas_call(
        _linear_kstream_kernel,
        out_shape=jax.ShapeDtypeStruct((m, n), jnp.float32),
        grid=(num_m, num_k),
        in_specs=[
            pl.BlockSpec((bm, bk), lambda i, kk: (i, kk)),
            pl.BlockSpec((bk, n), lambda i, kk: (kk, 0)),
            pl.BlockSpec((1, n), lambda i, kk: (0, 0)),
        ],
        out_specs=pl.BlockSpec((bm, n), lambda i, kk: (i, 0)),
        compiler_params=pltpu.CompilerParams(
            dimension_semantics=("parallel", "arbitrary"),
            vmem_limit_bytes=60 << 20,
        ),
        cost_estimate=pl.CostEstimate(
            flops=2 * m * k * n,
            transcendentals=0,
            bytes_accessed=4 * (m * k + k * n + m * n + n),
        ),
    )(x2d, weight, bias.reshape(1, n))


def kernel(x, weight, bias):
    orig_shape = x.shape
    in_features, out_features = weight.shape
    x2d = x.reshape(-1, in_features).astype(jnp.float32)
    out = _linear(
        x2d,
        weight.astype(jnp.float32),
        bias.astype(jnp.float32),
        num_m=2,
        num_k=4,
    )
    return out.reshape(*orig_shape[:-1], out_features)


# speedup vs baseline: 1.0007x; 1.0007x over previous
import functools

import jax
import jax.numpy as jnp
from jax.experimental import pallas as pl
from jax.experimental.pallas import tpu as pltpu


def _linear_block_kernel(x_ref, w_ref, b_ref, o_ref):
    o_ref[...] = (
        jnp.dot(x_ref[...], w_ref[...], preferred_element_type=jnp.float32)
        + b_ref[...]
    )


@functools.partial(jax.jit, static_argnames=("num_n", "block_m"))
def _linear(x2d, weight, bias, *, num_n, block_m):
    m, k = x2d.shape
    _, n = weight.shape
    bn = n // num_n

    return pl.pallas_call(
        _linear_block_kernel,
        out_shape=jax.ShapeDtypeStruct((m, n), jnp.float32),
        grid=(num_n, m // block_m),
        in_specs=[
            pl.BlockSpec((block_m, k), lambda j, i: (i, 0)),
            pl.BlockSpec((k, bn), lambda j, i: (0, j)),
            pl.BlockSpec((1, bn), lambda j, i: (0, j)),
        ],
        out_specs=pl.BlockSpec((block_m, bn), lambda j, i: (i, j)),
        compiler_params=pltpu.CompilerParams(
            dimension_semantics=("parallel", "parallel"),
            vmem_limit_bytes=60 << 20,
        ),
        cost_estimate=pl.CostEstimate(
            flops=2 * m * k * n,
            transcendentals=0,
            bytes_accessed=4 * (2 * m * k + k * n + m * n + n),
        ),
    )(x2d, weight, bias.reshape(1, n))


def kernel(x, weight, bias):
    orig_shape = x.shape
    in_features, out_features = weight.shape
    x2d = x.reshape(-1, in_features).astype(jnp.float32)
    out = _linear(
        x2d,
        weight.astype(jnp.float32),
        bias.astype(jnp.float32),
        num_n=2,
        block_m=512,
    )
    return out.reshape(*orig_shape[:-1], out_features)


# speedup vs baseline: 1.0488x; 1.0481x over previous
import functools

import jax
import jax.numpy as jnp
from jax.experimental import pallas as pl
from jax.experimental.pallas import tpu as pltpu

_NK = 4


def _w_chunk_copy(w_hbm, w_vmem, sems, kc, bk):
    return pltpu.make_async_copy(
        w_hbm.at[pl.ds(kc * bk, bk), :],
        w_vmem.at[pl.ds(kc * bk, bk), :],
        sems.at[kc],
    )


def _linear_kernel(x_ref, w_hbm, b_ref, o_ref, w_vmem, sems):
    t = pl.program_id(1)
    k = w_vmem.shape[0]
    bk = k // _NK

    @pl.when(t == 0)
    def _first_step():
        for kc in range(_NK):
            _w_chunk_copy(w_hbm, w_vmem, sems, kc, bk).start()
        o_ref[...] = jnp.broadcast_to(b_ref[...], o_ref.shape)
        for kc in range(_NK):
            _w_chunk_copy(w_hbm, w_vmem, sems, kc, bk).wait()
            o_ref[...] += jnp.dot(
                x_ref[:, kc * bk:(kc + 1) * bk],
                w_vmem[kc * bk:(kc + 1) * bk, :],
                preferred_element_type=jnp.float32,
            )

    @pl.when(t != 0)
    def _steady_step():
        o_ref[...] = (
            jnp.dot(x_ref[...], w_vmem[...], preferred_element_type=jnp.float32)
            + b_ref[...]
        )


@functools.partial(jax.jit, static_argnames=("num_cores", "block_m"))
def _linear(x2d, weight, bias, *, num_cores, block_m):
    m, k = x2d.shape
    _, n = weight.shape
    steps = m // (num_cores * block_m)

    return pl.pallas_call(
        _linear_kernel,
        out_shape=jax.ShapeDtypeStruct((m, n), jnp.float32),
        grid=(num_cores, steps),
        in_specs=[
            pl.BlockSpec((block_m, k), lambda i, t, s=steps: (i * s + t, 0)),
            pl.BlockSpec(memory_space=pl.ANY),
            pl.BlockSpec((1, n), lambda i, t: (0, 0)),
        ],
        out_specs=pl.BlockSpec((block_m, n), lambda i, t, s=steps: (i * s + t, 0)),
        scratch_shapes=[
            pltpu.VMEM((k, n), jnp.float32),
            pltpu.SemaphoreType.DMA((_NK,)),
        ],
        compiler_params=pltpu.CompilerParams(
            dimension_semantics=("parallel", "arbitrary"),
            vmem_limit_bytes=60 << 20,
        ),
        cost_estimate=pl.CostEstimate(
            flops=2 * m * k * n,
            transcendentals=0,
            bytes_accessed=4 * (m * k + k * n + m * n + n),
        ),
    )(x2d, weight, bias.reshape(1, n))


def kernel(x, weight, bias):
    orig_shape = x.shape
    in_features, out_features = weight.shape
    x2d = x.reshape(-1, in_features).astype(jnp.float32)
    out = _linear(
        x2d,
        weight.astype(jnp.float32),
        bias.astype(jnp.float32),
        num_cores=2,
        block_m=512,
    )
    return out.reshape(*orig_shape[:-1], out_features)


# speedup vs baseline: 1.1466x; 1.0932x over previous
import functools

import jax
import jax.numpy as jnp
from jax.experimental import pallas as pl
from jax.experimental.pallas import tpu as pltpu


def _linear_block_kernel(x_ref, w_ref, b_ref, o_ref):
    o_ref[...] = (
        jnp.dot(x_ref[...], w_ref[...], preferred_element_type=jnp.float32)
        + b_ref[...]
    )


@functools.partial(jax.jit, static_argnames=("block_m",))
def _linear(x2d, weight, bias, *, block_m):
    m, k = x2d.shape
    _, n = weight.shape
    grid = (m // block_m,)

    return pl.pallas_call(
        _linear_block_kernel,
        out_shape=jax.ShapeDtypeStruct((m, n), jnp.float32),
        grid=grid,
        in_specs=[
            pl.BlockSpec((block_m, k), lambda i: (i, 0)),
            pl.BlockSpec((k, n), lambda i: (0, 0)),
            pl.BlockSpec((1, n), lambda i: (0, 0)),
        ],
        out_specs=pl.BlockSpec((block_m, n), lambda i: (i, 0)),
        compiler_params=pltpu.CompilerParams(
            dimension_semantics=("parallel",),
            vmem_limit_bytes=60 << 20,
        ),
        cost_estimate=pl.CostEstimate(
            flops=2 * m * k * n,
            transcendentals=0,
            bytes_accessed=4 * (m * k + k * n + m * n + n),
        ),
    )(x2d, weight, bias.reshape(1, n))


def kernel(x, weight, bias):
    orig_shape = x.shape
    in_features, out_features = weight.shape
    x2d = x.reshape(-1, in_features).astype(jnp.float32)
    out = _linear(
        x2d,
        weight.astype(jnp.float32),
        bias.astype(jnp.float32),
        block_m=512,
    )
    return out.reshape(*orig_shape[:-1], out_features)
